# Initial kernel scaffold; baseline (speedup 1.0000x reference)
import jax
import jax.numpy as jnp
from jax.experimental import pallas as pl


def kernel(x, w_eff, b_eff, scale):
    raise NotImplementedError("write your pallas kernel here")


# speedup vs baseline: 6.4515x; 6.4515x over previous
import jax
import jax.numpy as jnp
from jax.experimental import pallas as pl
from jax.experimental.pallas import tpu as pltpu


def _matmul_kernel(x_ref, w_ref, b_ref, s_ref, y_ref, acc_ref):
    k = pl.program_id(0)

    @pl.when(k == 0)
    def _():
        acc_ref[...] = jnp.zeros_like(acc_ref)

    acc_ref[...] += jnp.dot(
        x_ref[...].astype(jnp.bfloat16),
        w_ref[...].astype(jnp.bfloat16),
        preferred_element_type=jnp.float32,
    )

    @pl.when(k == pl.num_programs(0) - 1)
    def _():
        y_ref[...] = ((acc_ref[...] + b_ref[...]) * s_ref[...]).astype(y_ref.dtype)


def kernel(x, w_eff, b_eff, scale):
    bs, n_tok, d_model = x.shape
    t_dim, _, i_pad = w_eff.shape
    k_total = n_tok * d_model

    x2 = x.reshape(bs, k_total)
    w2 = w_eff.reshape(k_total, i_pad)

    bk = 4096
    while k_total % bk:
        bk //= 2
    n_k = k_total // bk

    return pl.pallas_call(
        _matmul_kernel,
        out_shape=jax.ShapeDtypeStruct((bs, i_pad), x.dtype),
        grid=(n_k,),
        in_specs=[
            pl.BlockSpec((bs, bk), lambda k: (0, k)),
            pl.BlockSpec((bk, i_pad), lambda k: (k, 0)),
            pl.BlockSpec((1, i_pad), lambda k: (0, 0)),
            pl.BlockSpec((1, i_pad), lambda k: (0, 0)),
        ],
        out_specs=pl.BlockSpec((bs, i_pad), lambda k: (0, 0)),
        scratch_shapes=[pltpu.VMEM((bs, i_pad), jnp.float32)],
        compiler_params=pltpu.CompilerParams(
            dimension_semantics=("arbitrary",),
            vmem_limit_bytes=60 * 1024 * 1024,
        ),
    )(x2, w2, b_eff, scale)


# speedup vs baseline: 10.8010x; 1.6742x over previous
import jax
import jax.numpy as jnp
from jax.experimental import pallas as pl
from jax.experimental.pallas import tpu as pltpu


def _matmul_kernel(x_ref, w_ref, b_ref, s_ref, y_ref, acc_ref):
    k = pl.program_id(0)

    @pl.when(k == 0)
    def _():
        acc_ref[...] = jnp.zeros_like(acc_ref)

    tk = x_ref.shape[1]
    acc = acc_ref[...]
    for i in range(tk):
        acc += jnp.dot(
            x_ref[:, i, :].astype(jnp.bfloat16),
            w_ref[i].astype(jnp.bfloat16),
            preferred_element_type=jnp.float32,
        )
    acc_ref[...] = acc

    @pl.when(k == pl.num_programs(0) - 1)
    def _():
        y_ref[...] = ((acc_ref[...] + b_ref[...]) * s_ref[...]).astype(y_ref.dtype)


def kernel(x, w_eff, b_eff, scale):
    bs, n_tok, d_model = x.shape
    t_dim, _, i_pad = w_eff.shape

    tk = 16
    while n_tok % tk:
        tk //= 2
    n_k = n_tok // tk

    return pl.pallas_call(
        _matmul_kernel,
        out_shape=jax.ShapeDtypeStruct((bs, i_pad), x.dtype),
        grid=(n_k,),
        in_specs=[
            pl.BlockSpec((bs, tk, d_model), lambda k: (0, k, 0)),
            pl.BlockSpec((tk, d_model, i_pad), lambda k: (k, 0, 0)),
            pl.BlockSpec((1, i_pad), lambda k: (0, 0)),
            pl.BlockSpec((1, i_pad), lambda k: (0, 0)),
        ],
        out_specs=pl.BlockSpec((bs, i_pad), lambda k: (0, 0)),
        scratch_shapes=[pltpu.VMEM((bs, i_pad), jnp.float32)],
        compiler_params=pltpu.CompilerParams(
            dimension_semantics=("arbitrary",),
            vmem_limit_bytes=60 * 1024 * 1024,
        ),
    )(x, w_eff, b_eff, scale)
